# baseline (device time: 50632 ns/iter reference)
import jax
import jax.numpy as jnp
from jax import lax
from jax.experimental import pallas as pl
from jax.experimental.pallas import tpu as pltpu

N_DEV = 4
_GELU_C = 0.7978845608028654
_RT = 4


def _gelu(y):
    return 0.5 * y * (1.0 + jnp.tanh(_GELU_C * (y + 0.044715 * y * y * y)))


def kernel(x, w_mat):
    m_per, k = x.shape
    _, n = w_mat.shape
    n_per = n // N_DEV
    rt = m_per // _RT
    hm = m_per // 2

    offs = (2, 1, 3, 0)

    def body(
        x_hbm,
        w_hbm,
        out_hbm,
        x_land,
        x_bf,
        w_land,
        w_bf,
        send_buf,
        recv_buf,
        stage,
        x_sems,
        w_sems,
        out_sems,
        send_sems,
        recv_sems,
    ):
        my = lax.axis_index("i")

        def x_copy(i):
            return pltpu.make_async_copy(
                x_hbm.at[pl.ds(i * rt, rt), :],
                x_land.at[i % 2],
                x_sems.at[i % 2],
            )

        def w_copy(j):
            t = (my + offs[j]) % N_DEV
            return pltpu.make_async_copy(
                w_hbm.at[:, pl.ds(t * n_per, n_per)],
                w_land.at[j % 2],
                w_sems.at[j % 2],
            )

        xc = [x_copy(i) for i in range(_RT)]
        wc = [w_copy(j) for j in range(4)]
        xc[0].start()
        wc[0].start()
        xc[1].start()

        barrier = pltpu.get_barrier_semaphore()
        for d in range(1, N_DEV):
            pl.semaphore_signal(
                barrier,
                inc=1,
                device_id=((my + d) % N_DEV,),
                device_id_type=pl.DeviceIdType.MESH,
            )
        pl.semaphore_wait(barrier, N_DEV - 1)

        def xwait(i):
            xc[i].wait()
            x_bf[pl.ds(i * rt, rt), :] = x_land[i % 2].astype(jnp.bfloat16)

        def wwait(j):
            wc[j].wait()
            w_bf[min(j, 1)] = w_land[j % 2].astype(jnp.bfloat16)

        rdmas = {}

        def emit_rdma(j, rows, key):
            rdma = pltpu.make_async_remote_copy(
                src_ref=send_buf.at[j, rows, :],
                dst_ref=recv_buf.at[j, rows, :],
                send_sem=send_sems.at[key],
                recv_sem=recv_sems.at[key],
                device_id=((my + offs[j]) % N_DEV,),
                device_id_type=pl.DeviceIdType.MESH,
            )
            rdma.start()
            rdmas[key] = rdma

        def sub(j, lo, sz, key):
            rows = pl.ds(lo, sz)
            y = _gelu(
                jnp.dot(
                    x_bf[rows, :],
                    w_bf[min(j, 1)],
                    preferred_element_type=jnp.float32,
                )
            )
            if offs[j] == 0:
                stage[0, rows, :] = y
            else:
                send_buf[j, rows, :] = y.astype(jnp.bfloat16)
                emit_rdma(j, rows, key)

        out_copies = [None, None]

        def drain(j, slot):
            src = (my - offs[j]) % N_DEV
            if out_copies[slot] is not None:
                out_copies[slot].wait()
            nsub = _RT if j == 0 else 2
            step = rt if j == 0 else hm
            for r in range(nsub):
                rows = pl.ds(r * step, step)
                rdmas[(j, r)].wait_recv()
                stage[slot, rows, :] = recv_buf[j, rows, :].astype(jnp.float32)
            oc = pltpu.make_async_copy(
                stage.at[slot],
                out_hbm.at[pl.ds(src * m_per, m_per), :],
                out_sems.at[slot],
            )
            oc.start()
            out_copies[slot] = oc

        xwait(0)
        wwait(0)
        sub(0, 0 * rt, rt, (0, 0))
        xwait(1)
        xc[2].start()
        xc[3].start()
        wc[1].start()
        sub(0, 1 * rt, rt, (0, 1))
        xwait(2)
        sub(0, 2 * rt, rt, (0, 2))
        xwait(3)
        sub(0, 3 * rt, rt, (0, 3))
        wwait(1)
        wc[2].start()
        sub(1, 0, hm, (1, 0))
        sub(1, hm, hm, (1, 1))
        wwait(2)
        wc[3].start()
        sub(2, 0, hm, (2, 0))
        sub(2, hm, hm, (2, 1))
        drain(0, 1)
        wwait(3)
        sub(3, 0, hm, None)
        sub(3, hm, hm, None)
        oc = pltpu.make_async_copy(
            stage.at[0],
            out_hbm.at[pl.ds(my * m_per, m_per), :],
            out_sems.at[0],
        )
        oc.start()
        out_copies[0] = oc
        drain(1, 1)
        drain(2, 0)

        out_copies[0].wait()
        out_copies[1].wait()
        for rd in rdmas.values():
            rd.wait_send()

    out_shape = jax.ShapeDtypeStruct((N_DEV * m_per, n_per), jnp.float32)
    return pl.pallas_call(
        body,
        out_shape=out_shape,
        in_specs=[
            pl.BlockSpec(memory_space=pl.ANY),
            pl.BlockSpec(memory_space=pl.ANY),
        ],
        out_specs=pl.BlockSpec(memory_space=pl.ANY),
        scratch_shapes=[
            pltpu.VMEM((2, rt, k), jnp.float32),
            pltpu.VMEM((m_per, k), jnp.bfloat16),
            pltpu.VMEM((2, k, n_per), jnp.float32),
            pltpu.VMEM((2, k, n_per), jnp.bfloat16),
            pltpu.VMEM((3, m_per, n_per), jnp.bfloat16),
            pltpu.VMEM((3, m_per, n_per), jnp.bfloat16),
            pltpu.VMEM((2, m_per, n_per), jnp.float32),
            pltpu.SemaphoreType.DMA((2,)),
            pltpu.SemaphoreType.DMA((2,)),
            pltpu.SemaphoreType.DMA((2,)),
            pltpu.SemaphoreType.DMA((3, _RT)),
            pltpu.SemaphoreType.DMA((3, _RT)),
        ],
        compiler_params=pltpu.CompilerParams(
            collective_id=0, vmem_limit_bytes=100 * 1024 * 1024
        ),
    )(x, w_mat)


# device time: 42167 ns/iter; 1.2007x vs baseline; 1.2007x over previous
import jax
import jax.numpy as jnp
from jax import lax
from jax.experimental import pallas as pl
from jax.experimental.pallas import tpu as pltpu

N_DEV = 4
_GELU_C = 0.7978845608028654
_RT = 4
_WH = 2


def _gelu(y):
    return 0.5 * y * (1.0 + jnp.tanh(_GELU_C * (y + 0.044715 * y * y * y)))


def kernel(x, w_mat):
    m_per, k = x.shape
    _, n = w_mat.shape
    n_per = n // N_DEV
    rt = m_per // _RT
    wh = n_per // _WH

    offs = (2, 1, 3, 0)

    def body(
        x_hbm,
        w_hbm,
        out_hbm,
        x_land,
        x_bf,
        w_land,
        w_bf,
        send_buf,
        recv_buf,
        stage,
        x_sems,
        w_sems,
        out_sems,
        send_sems,
        recv_sems,
    ):
        my = lax.axis_index("i")

        def x_copy(i):
            return pltpu.make_async_copy(
                x_hbm.at[pl.ds(i * rt, rt), :],
                x_land.at[i % 2],
                x_sems.at[i % 2],
            )

        def w_slot(j, h):
            return (j * _WH + h) % 4

        def w_copy(j, h):
            t = (my + offs[j]) % N_DEV
            s = w_slot(j, h)
            return pltpu.make_async_copy(
                w_hbm.at[:, pl.ds(t * n_per + h * wh, wh)],
                w_land.at[s],
                w_sems.at[s],
            )

        xc = [x_copy(i) for i in range(_RT)]
        wc = {(j, h): w_copy(j, h) for j in range(4) for h in range(_WH)}
        xc[0].start()
        wc[(0, 0)].start()
        xc[1].start()
        wc[(0, 1)].start()

        barrier = pltpu.get_barrier_semaphore()
        for d in range(1, N_DEV):
            pl.semaphore_signal(
                barrier,
                inc=1,
                device_id=((my + d) % N_DEV,),
                device_id_type=pl.DeviceIdType.MESH,
            )
        pl.semaphore_wait(barrier, N_DEV - 1)

        def xwait(i, issue=()):
            xc[i].wait()
            x_bf[pl.ds(i * rt, rt), :] = x_land[i % 2].astype(jnp.bfloat16)
            for nxt in issue:
                xc[nxt].start()

        def wwait(j, h, issue=()):
            wc[(j, h)].wait()
            w_bf[min(j, 1), :, pl.ds(h * wh, wh)] = w_land[
                w_slot(j, h)
            ].astype(jnp.bfloat16)
            for nxt in issue:
                wc[nxt].start()

        rdmas = {}

        def emit_rdma(j, rows, cols, key):
            rdma = pltpu.make_async_remote_copy(
                src_ref=send_buf.at[j, rows, cols],
                dst_ref=recv_buf.at[j, rows, cols],
                send_sem=send_sems.at[key],
                recv_sem=recv_sems.at[key],
                device_id=((my + offs[j]) % N_DEV,),
                device_id_type=pl.DeviceIdType.MESH,
            )
            rdma.start()
            rdmas[key] = rdma

        def subq(r, h):
            rows = pl.ds(r * rt, rt)
            cols = pl.ds(h * wh, wh)
            y = _gelu(
                jnp.dot(
                    x_bf[rows, :],
                    w_bf[0, :, cols],
                    preferred_element_type=jnp.float32,
                )
            )
            send_buf[0, rows, cols] = y.astype(jnp.bfloat16)
            emit_rdma(0, rows, cols, (0, r, h))

        def sub(r, j):
            rows = pl.ds(r * rt, rt)
            y = _gelu(
                jnp.dot(
                    x_bf[rows, :],
                    w_bf[1],
                    preferred_element_type=jnp.float32,
                )
            )
            if offs[j] == 0:
                stage[0, rows, :] = y
            else:
                send_buf[j, rows, :] = y.astype(jnp.bfloat16)
                emit_rdma(j, rows, slice(None), (j, r, 0))

        out_copies = [None, None]

        def drain(j, slot):
            src = (my - offs[j]) % N_DEV
            if out_copies[slot] is not None:
                out_copies[slot].wait()
            for r in range(_RT):
                rows = pl.ds(r * rt, rt)
                if j == 0:
                    for h in range(_WH):
                        cols = pl.ds(h * wh, wh)
                        rdmas[(j, r, h)].wait_recv()
                        stage[slot, rows, cols] = recv_buf[
                            j, rows, cols
                        ].astype(jnp.float32)
                else:
                    rdmas[(j, r, 0)].wait_recv()
                    stage[slot, rows, :] = recv_buf[j, rows, :].astype(
                        jnp.float32
                    )
            oc = pltpu.make_async_copy(
                stage.at[slot],
                out_hbm.at[pl.ds(src * m_per, m_per), :],
                out_sems.at[slot],
            )
            oc.start()
            out_copies[slot] = oc

        xwait(0, issue=(2,))
        xwait(1, issue=(3,))
        wwait(0, 0, issue=((1, 0),))
        subq(0, 0)
        subq(1, 0)
        wwait(0, 1, issue=((1, 1),))
        subq(0, 1)
        subq(1, 1)
        xwait(2)
        subq(2, 0)
        subq(2, 1)
        xwait(3)
        subq(3, 0)
        subq(3, 1)
        wwait(1, 0, issue=((2, 0),))
        wwait(1, 1, issue=((2, 1),))
        for r in range(_RT):
            sub(r, 1)
        wwait(2, 0, issue=((3, 0),))
        wwait(2, 1, issue=((3, 1),))
        for r in range(_RT):
            sub(r, 2)
        drain(0, 1)
        wwait(3, 0)
        wwait(3, 1)
        for r in range(_RT):
            sub(r, 3)
        oc = pltpu.make_async_copy(
            stage.at[0],
            out_hbm.at[pl.ds(my * m_per, m_per), :],
            out_sems.at[0],
        )
        oc.start()
        out_copies[0] = oc
        drain(1, 1)
        drain(2, 0)

        out_copies[0].wait()
        out_copies[1].wait()
        for rd in rdmas.values():
            rd.wait_send()

    out_shape = jax.ShapeDtypeStruct((N_DEV * m_per, n_per), jnp.float32)
    return pl.pallas_call(
        body,
        out_shape=out_shape,
        in_specs=[
            pl.BlockSpec(memory_space=pl.ANY),
            pl.BlockSpec(memory_space=pl.ANY),
        ],
        out_specs=pl.BlockSpec(memory_space=pl.ANY),
        scratch_shapes=[
            pltpu.VMEM((2, rt, k), jnp.float32),
            pltpu.VMEM((m_per, k), jnp.bfloat16),
            pltpu.VMEM((4, k, wh), jnp.float32),
            pltpu.VMEM((2, k, n_per), jnp.bfloat16),
            pltpu.VMEM((3, m_per, n_per), jnp.bfloat16),
            pltpu.VMEM((3, m_per, n_per), jnp.bfloat16),
            pltpu.VMEM((2, m_per, n_per), jnp.float32),
            pltpu.SemaphoreType.DMA((2,)),
            pltpu.SemaphoreType.DMA((4,)),
            pltpu.SemaphoreType.DMA((2,)),
            pltpu.SemaphoreType.DMA((3, _RT, _WH)),
            pltpu.SemaphoreType.DMA((3, _RT, _WH)),
        ],
        compiler_params=pltpu.CompilerParams(
            collective_id=0, vmem_limit_bytes=100 * 1024 * 1024
        ),
    )(x, w_mat)
